# baseline (speedup 1.0000x reference)
import functools

import jax
import jax.numpy as jnp
from jax import lax
from jax.experimental import pallas as pl
from jax.experimental.pallas import tpu as pltpu
from jax.experimental.pallas import tpu_sc as plsc

N = 10000
D_IN, D_H, D_OUT = 128, 128, 64
E = 320000

N_PAD = 10240
NTILES = 16
NCORES = 2
NW = NCORES * NTILES
CH = 128
NCH = 80
EPW = NCH * CH
E_PAD = NW * EPW
NPAIR = NCH // 2
RPT = N_PAD // NTILES
D_DEG = 128
BLK = 512
GRID = N_PAD // BLK


def _fill(ref, width, value):
    per_row = width // 16

    def body(i, _):
        r = i // per_row
        col = (i % per_row) * 16
        ref[r, pl.ds(col, 16)] = jnp.full((16,), value, jnp.float32)
        return 0

    lax.fori_loop(0, ref.shape[0] * per_row, body, 0)


def _make_agg(D, chw=128, nbuf=2, nch0=80, nch1=80):
    mesh = plsc.VectorSubcoreMesh(core_axis_name="c", subcore_axis_name="s")
    assert (nch0 + nch1) * chw * NTILES == E_PAD

    def body(p_hbm, src_hbm, dst_hbm, acc_out, *rest):
        sidx = rest[0:nbuf]
        didx = rest[nbuf:2 * nbuf]
        rows = rest[2 * nbuf:3 * nbuf]
        acc = rest[3 * nbuf]
        gsem = rest[3 * nbuf + 1:4 * nbuf + 1]
        c = lax.axis_index("c")
        s = lax.axis_index("s")
        wid = c * NTILES + s

        _fill(rows[0], D, 0.0)
        base_r = s * RPT
        for j in range(RPT // chw):
            pltpu.sync_copy(rows[0], acc.at[pl.ds(base_r + j * chw, chw)])
        plsc.subcore_barrier()

        ebase = jnp.where(c == 0, s * nch0, NTILES * nch0 + s * nch1) * chw
        ngrp = jnp.where(c == 0, nch0 // nbuf, nch1 // nbuf)

        def load_idx(off, b):
            pltpu.sync_copy(src_hbm.at[pl.ds(off, chw)], sidx[b])
            pltpu.sync_copy(dst_hbm.at[pl.ds(off, chw)], didx[b])

        for b in range(nbuf):
            load_idx(ebase + b * chw, b)
            pltpu.async_copy(p_hbm.at[sidx[b]], rows[b], gsem[b])

        def step(j, _):
            i0 = nbuf * j
            for b in range(nbuf):
                pltpu.make_async_copy(p_hbm.at[sidx[b]], rows[b],
                                      gsem[b]).wait()
                pltpu.sync_copy(rows[b], acc.at[didx[b]], add=True)
                load_idx(ebase + (i0 + b + nbuf) * chw, b)
                pltpu.async_copy(p_hbm.at[sidx[b]], rows[b], gsem[b])
            return 0

        lax.fori_loop(0, ngrp, step, 0)
        for b in range(nbuf):
            pltpu.make_async_copy(p_hbm.at[sidx[b]], rows[b], gsem[b]).wait()
        plsc.subcore_barrier()

        for j in range(RPT // chw):
            r0 = base_r + j * chw
            pltpu.sync_copy(acc.at[pl.ds(r0, chw)], rows[0])
            pltpu.sync_copy(rows[0], acc_out.at[pl.ds(c * N_PAD + r0, chw)])

    return pl.kernel(
        body,
        out_type=jax.ShapeDtypeStruct((NCORES * N_PAD, D), jnp.float32),
        mesh=mesh,
        scratch_types=(
            tuple(pltpu.VMEM((chw,), jnp.int32) for _ in range(2 * nbuf))
            + tuple(pltpu.VMEM((chw, D), jnp.float32) for _ in range(nbuf))
            + (pltpu.VMEM_SHARED((N_PAD, D), jnp.float32),)
            + tuple(pltpu.SemaphoreType.DMA for _ in range(nbuf))
        ))


NRP = 10000
HALF = N_PAD // 2
ABASE = NRP
TRASH = HALF
CH2 = 64
EPT = E_PAD // NTILES
NCH2 = EPT // CH2


def _make_agg_spm(D):
    mesh = plsc.VectorSubcoreMesh(core_axis_name="c", subcore_axis_name="s")

    def body(p_hbm, src_hbm, dst_hbm, acc_out, sidx, didx, rows, spm, gsem):
        c = lax.axis_index("c")
        s = lax.axis_index("s")

        for k in range(16):
            b = s + 16 * k

            @pl.when(b < NRP // 40)
            def _():
                pltpu.sync_copy(p_hbm.at[pl.ds(b * 40, 40)],
                                rows.at[pl.ds(0, 40)])
                pltpu.sync_copy(rows.at[pl.ds(0, 40)],
                                spm.at[pl.ds(b * 40, 40)])

        _fill(rows, D, 0.0)
        for j in range(HALF // NTILES // CH2):
            r0 = ABASE + s * (HALF // NTILES) + j * CH2
            pltpu.sync_copy(rows, spm.at[pl.ds(r0, CH2)])

        @pl.when(s == 0)
        def _():
            pltpu.sync_copy(rows.at[pl.ds(0, 8)],
                            spm.at[pl.ds(ABASE + HALF, 8)])
        plsc.subcore_barrier()

        ebase = s * EPT
        nbase = c * HALF

        def step(i, _):
            off = ebase + i * CH2
            pltpu.sync_copy(src_hbm.at[pl.ds(off, CH2)], sidx)
            pltpu.sync_copy(dst_hbm.at[pl.ds(off, CH2)], didx)
            for j in range(CH2 // 16):
                d = didx[pl.ds(j * 16, 16)]
                keep = (d >= nbase) & (d < nbase + HALF)
                didx[pl.ds(j * 16, 16)] = (
                    jnp.where(keep, d - nbase, TRASH) + ABASE)
            pltpu.async_copy(spm.at[sidx], rows, gsem).wait()
            pltpu.sync_copy(rows, spm.at[didx], add=True)
            return 0

        lax.fori_loop(0, NCH2, step, 0)
        plsc.subcore_barrier()

        rpt2 = HALF // NTILES
        for j in range(rpt2 // CH2):
            r0 = s * rpt2 + j * CH2
            pltpu.sync_copy(spm.at[pl.ds(ABASE + r0, CH2)], rows)
            pltpu.sync_copy(rows, acc_out.at[pl.ds(c * HALF + r0, CH2)])

    return pl.kernel(
        body,
        out_type=jax.ShapeDtypeStruct((N_PAD, D), jnp.float32),
        mesh=mesh,
        scratch_types=(
            pltpu.VMEM((CH2,), jnp.int32),
            pltpu.VMEM((CH2,), jnp.int32),
            pltpu.VMEM((CH2, D), jnp.float32),
            pltpu.VMEM_SHARED((NRP + HALF + 8, D), jnp.float32),
            pltpu.SemaphoreType.DMA,
        ))


def _make_deg(W=D_DEG):
    mesh = plsc.VectorSubcoreMesh(core_axis_name="c", subcore_axis_name="s")

    def body(src_hbm, dst_hbm, deg_out, didx_a, didx_b, ones, acc):
        c = lax.axis_index("c")
        s = lax.axis_index("s")
        wid = c * NTILES + s

        _fill(ones, W, 0.0)
        base_r = s * RPT
        for j in range(RPT // CH):
            pltpu.sync_copy(ones, acc.at[pl.ds(base_r + j * CH, CH)])
        _fill(ones, W, 1.0)
        plsc.subcore_barrier()

        ebase = wid * EPW
        pltpu.sync_copy(dst_hbm.at[pl.ds(ebase, CH)], didx_a)

        def step(j, _):
            i0 = 2 * j
            pltpu.sync_copy(dst_hbm.at[pl.ds(ebase + (i0 + 1) * CH, CH)],
                            didx_b)
            pltpu.sync_copy(ones, acc.at[didx_a], add=True)
            pltpu.sync_copy(dst_hbm.at[pl.ds(ebase + (i0 + 2) * CH, CH)],
                            didx_a)
            pltpu.sync_copy(ones, acc.at[didx_b], add=True)
            return 0

        lax.fori_loop(0, NPAIR, step, 0)
        plsc.subcore_barrier()

        for j in range(RPT // CH):
            r0 = base_r + j * CH
            pltpu.sync_copy(acc.at[pl.ds(r0, CH)], ones)
            pltpu.sync_copy(ones, deg_out.at[pl.ds(c * N_PAD + r0, CH)])

    return pl.kernel(
        body,
        out_type=jax.ShapeDtypeStruct((NCORES * N_PAD, W), jnp.float32),
        mesh=mesh,
        scratch_types=(
            pltpu.VMEM((CH,), jnp.int32),
            pltpu.VMEM((CH,), jnp.int32),
            pltpu.VMEM((CH, W), jnp.float32),
            pltpu.VMEM_SHARED((N_PAD, W), jnp.float32),
        ))


_agg_h = _make_agg_spm(D_H)
_deg_k = _make_deg()


def _pre_body(x_ref, wl_ref, wr_ref, bl_ref, p_ref, r_ref):
    h = x_ref[...]
    p_ref[...] = jnp.dot(h, wl_ref[...], preferred_element_type=jnp.float32)
    r_ref[...] = (jnp.dot(h, wr_ref[...], preferred_element_type=jnp.float32)
                  + bl_ref[...])


def _pre(x, wlT, wrT, bl):
    d_in, d_o = wlT.shape
    return pl.pallas_call(
        _pre_body,
        grid=(GRID,),
        in_specs=[
            pl.BlockSpec((BLK, d_in), lambda i: (i, 0)),
            pl.BlockSpec((d_in, d_o), lambda i: (0, 0)),
            pl.BlockSpec((d_in, d_o), lambda i: (0, 0)),
            pl.BlockSpec((1, d_o), lambda i: (0, 0)),
        ],
        out_specs=[
            pl.BlockSpec((BLK, d_o), lambda i: (i, 0)),
            pl.BlockSpec((BLK, d_o), lambda i: (i, 0)),
        ],
        out_shape=[
            jax.ShapeDtypeStruct((N_PAD, d_o), jnp.float32),
            jax.ShapeDtypeStruct((N_PAD, d_o), jnp.float32),
        ],
    )(x, wlT, wrT, bl)


def _combine(acc_ref, dacc_ref, r_ref):
    deg = dacc_ref[0] + dacc_ref[1]
    mean = acc_ref[...] / jnp.maximum(deg, 1.0)
    return mean + r_ref[...]


def _mid_body(acc_ref, dacc_ref, r_ref, wl_ref, wr_ref, bl_ref, p_ref, rn_ref):
    h = jnp.maximum(_combine(acc_ref, dacc_ref, r_ref), 0.0)
    p_ref[...] = jnp.dot(h, wl_ref[...], preferred_element_type=jnp.float32)
    rn_ref[...] = (jnp.dot(h, wr_ref[...], preferred_element_type=jnp.float32)
                   + bl_ref[...])


def _mid(acc, dacc, r, wlT, wrT, bl):
    d, d_po = wlT.shape
    d_ro = wrT.shape[1]
    return pl.pallas_call(
        _mid_body,
        grid=(GRID,),
        in_specs=[
            pl.BlockSpec((BLK, d), lambda i: (i, 0)),
            pl.BlockSpec((NCORES, BLK, 1), lambda i: (0, i, 0)),
            pl.BlockSpec((BLK, d), lambda i: (i, 0)),
            pl.BlockSpec((d, d_po), lambda i: (0, 0)),
            pl.BlockSpec((d, d_ro), lambda i: (0, 0)),
            pl.BlockSpec((1, d_ro), lambda i: (0, 0)),
        ],
        out_specs=[
            pl.BlockSpec((BLK, d_po), lambda i: (i, 0)),
            pl.BlockSpec((BLK, d_ro), lambda i: (i, 0)),
        ],
        out_shape=[
            jax.ShapeDtypeStruct((N_PAD, d_po), jnp.float32),
            jax.ShapeDtypeStruct((N_PAD, d_ro), jnp.float32),
        ],
    )(acc, dacc, r, wlT, wrT, bl)


def _final_body(acc_ref, dacc_ref, r_ref, o_ref):
    d = r_ref.shape[1]
    deg = dacc_ref[0] + dacc_ref[1]
    mean = acc_ref[:, :d] / jnp.maximum(deg, 1.0)
    z = mean + r_ref[...]
    m = jnp.max(z, axis=1, keepdims=True)
    ez = jnp.exp(z - m)
    lse = jnp.log(jnp.sum(ez, axis=1, keepdims=True)) + m
    o_ref[...] = z - lse


def _final(acc, dacc, r):
    d = r.shape[1]
    return pl.pallas_call(
        _final_body,
        grid=(GRID,),
        in_specs=[
            pl.BlockSpec((BLK, acc.shape[1]), lambda i: (i, 0)),
            pl.BlockSpec((NCORES, BLK, 1), lambda i: (0, i, 0)),
            pl.BlockSpec((BLK, d), lambda i: (i, 0)),
        ],
        out_specs=pl.BlockSpec((BLK, d), lambda i: (i, 0)),
        out_shape=jax.ShapeDtypeStruct((N_PAD, d), jnp.float32),
    )(acc, dacc, r)


@jax.jit
def kernel(x, edge_index, Wl1, bl1, Wr1, Wl2, bl2, Wr2, Wl3, bl3, Wr3):
    x_pad = jnp.zeros((N_PAD, D_IN), jnp.float32).at[:N].set(x)
    pad_e = E_PAD + 2 * CH - E
    pad_dst = N + jnp.arange(pad_e, dtype=jnp.int32) % (N_PAD - N)
    src = jnp.concatenate([edge_index[0], jnp.zeros((pad_e,), jnp.int32)])
    dst = jnp.concatenate([edge_index[1], pad_dst])

    dacc = _deg_k(src, dst).reshape(NCORES, N_PAD, D_DEG)[:, :, :1]

    p1, r1 = _pre(x_pad, Wl1.T, Wr1.T, bl1[None])
    acc1 = _agg_h(p1, src, dst)
    p2, r2 = _mid(acc1, dacc, r1, Wl2.T, Wr2.T, bl2[None])
    acc2 = _agg_h(p2, src, dst)
    wl3T_pad = jnp.pad(Wl3.T, ((0, 0), (0, D_H - D_OUT)))
    p3, r3 = _mid(acc2, dacc, r2, wl3T_pad, Wr3.T, bl3[None])
    acc3 = _agg_h(p3, src, dst)
    out = _final(acc3, dacc, r3)
    return out[:N]


# speedup vs baseline: 1.2871x; 1.2871x over previous
import jax
import jax.numpy as jnp
from jax import lax
from jax.experimental import pallas as pl
from jax.experimental.pallas import tpu as pltpu
from jax.experimental.pallas import tpu_sc as plsc

N = 10000
D_IN, D_H, D_OUT = 128, 128, 64
E = 320000

N_PAD = 10240
NTILES = 16
NCORES = 2
NW = NCORES * NTILES
CH = 128
NCH = 80
EPW = NCH * CH
E_PAD = NW * EPW
NPAIR = NCH // 2
RPT = N_PAD // NTILES
D_DEG = 128
BLK = 512
GRID = N_PAD // BLK


def _fill(ref, width, value):
    per_row = width // 16

    def body(i, _):
        r = i // per_row
        col = (i % per_row) * 16
        ref[r, pl.ds(col, 16)] = jnp.full((16,), value, jnp.float32)
        return 0

    lax.fori_loop(0, ref.shape[0] * per_row, body, 0)


def _make_agg(D, chw=128, nbuf=2, nch0=80, nch1=80):
    mesh = plsc.VectorSubcoreMesh(core_axis_name="c", subcore_axis_name="s")
    assert (nch0 + nch1) * chw * NTILES == E_PAD

    def body(p_hbm, src_hbm, dst_hbm, acc_out, *rest):
        sidx = rest[0:nbuf]
        didx = rest[nbuf:2 * nbuf]
        rows = rest[2 * nbuf:3 * nbuf]
        acc = rest[3 * nbuf]
        gsem = rest[3 * nbuf + 1:4 * nbuf + 1]
        c = lax.axis_index("c")
        s = lax.axis_index("s")
        wid = c * NTILES + s

        _fill(rows[0], D, 0.0)
        base_r = s * RPT
        for j in range(RPT // chw):
            pltpu.sync_copy(rows[0], acc.at[pl.ds(base_r + j * chw, chw)])
        plsc.subcore_barrier()

        ebase = jnp.where(c == 0, s * nch0, NTILES * nch0 + s * nch1) * chw
        ngrp = jnp.where(c == 0, nch0 // nbuf, nch1 // nbuf)

        def load_idx(off, b):
            pltpu.sync_copy(src_hbm.at[pl.ds(off, chw)], sidx[b])
            pltpu.sync_copy(dst_hbm.at[pl.ds(off, chw)], didx[b])

        for b in range(nbuf):
            load_idx(ebase + b * chw, b)
            pltpu.async_copy(p_hbm.at[sidx[b]], rows[b], gsem[b])

        def step(j, _):
            i0 = nbuf * j
            for b in range(nbuf):
                pltpu.make_async_copy(p_hbm.at[sidx[b]], rows[b],
                                      gsem[b]).wait()
                pltpu.sync_copy(rows[b], acc.at[didx[b]], add=True)
                load_idx(ebase + (i0 + b + nbuf) * chw, b)
                pltpu.async_copy(p_hbm.at[sidx[b]], rows[b], gsem[b])
            return 0

        lax.fori_loop(0, ngrp, step, 0)
        for b in range(nbuf):
            pltpu.make_async_copy(p_hbm.at[sidx[b]], rows[b], gsem[b]).wait()
        plsc.subcore_barrier()

        for j in range(RPT // chw):
            r0 = base_r + j * chw
            pltpu.sync_copy(acc.at[pl.ds(r0, chw)], rows[0])
            pltpu.sync_copy(rows[0], acc_out.at[pl.ds(c * N_PAD + r0, chw)])

    return pl.kernel(
        body,
        out_type=jax.ShapeDtypeStruct((NCORES * N_PAD, D), jnp.float32),
        mesh=mesh,
        scratch_types=(
            tuple(pltpu.VMEM((chw,), jnp.int32) for _ in range(2 * nbuf))
            + tuple(pltpu.VMEM((chw, D), jnp.float32) for _ in range(nbuf))
            + (pltpu.VMEM_SHARED((N_PAD, D), jnp.float32),)
            + tuple(pltpu.SemaphoreType.DMA for _ in range(nbuf))
        ))


def _make_deg(W=D_DEG):
    mesh = plsc.VectorSubcoreMesh(core_axis_name="c", subcore_axis_name="s")

    def body(src_hbm, dst_hbm, deg_out, didx_a, didx_b, ones, acc):
        c = lax.axis_index("c")
        s = lax.axis_index("s")
        wid = c * NTILES + s

        _fill(ones, W, 0.0)
        base_r = s * RPT
        for j in range(RPT // CH):
            pltpu.sync_copy(ones, acc.at[pl.ds(base_r + j * CH, CH)])
        _fill(ones, W, 1.0)
        plsc.subcore_barrier()

        ebase = wid * EPW
        pltpu.sync_copy(dst_hbm.at[pl.ds(ebase, CH)], didx_a)

        def step(j, _):
            i0 = 2 * j
            pltpu.sync_copy(dst_hbm.at[pl.ds(ebase + (i0 + 1) * CH, CH)],
                            didx_b)
            pltpu.sync_copy(ones, acc.at[didx_a], add=True)
            pltpu.sync_copy(dst_hbm.at[pl.ds(ebase + (i0 + 2) * CH, CH)],
                            didx_a)
            pltpu.sync_copy(ones, acc.at[didx_b], add=True)
            return 0

        lax.fori_loop(0, NPAIR, step, 0)
        plsc.subcore_barrier()

        for j in range(RPT // CH):
            r0 = base_r + j * CH
            pltpu.sync_copy(acc.at[pl.ds(r0, CH)], ones)
            pltpu.sync_copy(ones, deg_out.at[pl.ds(c * N_PAD + r0, CH)])

    return pl.kernel(
        body,
        out_type=jax.ShapeDtypeStruct((NCORES * N_PAD, W), jnp.float32),
        mesh=mesh,
        scratch_types=(
            pltpu.VMEM((CH,), jnp.int32),
            pltpu.VMEM((CH,), jnp.int32),
            pltpu.VMEM((CH, W), jnp.float32),
            pltpu.VMEM_SHARED((N_PAD, W), jnp.float32),
        ))


_agg_h = _make_agg(D_H)
_deg_k = _make_deg()


def _pre_body(x_ref, wl_ref, wr_ref, bl_ref, p_ref, r_ref):
    h = x_ref[...]
    p_ref[...] = jnp.dot(h, wl_ref[...], preferred_element_type=jnp.float32)
    r_ref[...] = (jnp.dot(h, wr_ref[...], preferred_element_type=jnp.float32)
                  + bl_ref[...])


def _pre(x, wlT, wrT, bl):
    d_in, d_o = wlT.shape
    return pl.pallas_call(
        _pre_body,
        grid=(GRID,),
        in_specs=[
            pl.BlockSpec((BLK, d_in), lambda i: (i, 0)),
            pl.BlockSpec((d_in, d_o), lambda i: (0, 0)),
            pl.BlockSpec((d_in, d_o), lambda i: (0, 0)),
            pl.BlockSpec((1, d_o), lambda i: (0, 0)),
        ],
        out_specs=[
            pl.BlockSpec((BLK, d_o), lambda i: (i, 0)),
            pl.BlockSpec((BLK, d_o), lambda i: (i, 0)),
        ],
        out_shape=[
            jax.ShapeDtypeStruct((N_PAD, d_o), jnp.float32),
            jax.ShapeDtypeStruct((N_PAD, d_o), jnp.float32),
        ],
    )(x, wlT, wrT, bl)


def _combine(acc_ref, dacc_ref, r_ref):
    deg = dacc_ref[0] + dacc_ref[1]
    mean = (acc_ref[0] + acc_ref[1]) / jnp.maximum(deg, 1.0)
    return mean + r_ref[...]


def _mid_body(acc_ref, dacc_ref, r_ref, wl_ref, wr_ref, bl_ref, p_ref, rn_ref):
    h = jnp.maximum(_combine(acc_ref, dacc_ref, r_ref), 0.0)
    p_ref[...] = jnp.dot(h, wl_ref[...], preferred_element_type=jnp.float32)
    rn_ref[...] = (jnp.dot(h, wr_ref[...], preferred_element_type=jnp.float32)
                   + bl_ref[...])


def _mid(acc, dacc, r, wlT, wrT, bl):
    d, d_po = wlT.shape
    d_ro = wrT.shape[1]
    return pl.pallas_call(
        _mid_body,
        grid=(GRID,),
        in_specs=[
            pl.BlockSpec((NCORES, BLK, d), lambda i: (0, i, 0)),
            pl.BlockSpec((NCORES, BLK, 1), lambda i: (0, i, 0)),
            pl.BlockSpec((BLK, d), lambda i: (i, 0)),
            pl.BlockSpec((d, d_po), lambda i: (0, 0)),
            pl.BlockSpec((d, d_ro), lambda i: (0, 0)),
            pl.BlockSpec((1, d_ro), lambda i: (0, 0)),
        ],
        out_specs=[
            pl.BlockSpec((BLK, d_po), lambda i: (i, 0)),
            pl.BlockSpec((BLK, d_ro), lambda i: (i, 0)),
        ],
        out_shape=[
            jax.ShapeDtypeStruct((N_PAD, d_po), jnp.float32),
            jax.ShapeDtypeStruct((N_PAD, d_ro), jnp.float32),
        ],
    )(acc, dacc, r, wlT, wrT, bl)


def _final_body(acc_ref, dacc_ref, r_ref, o_ref):
    d = r_ref.shape[1]
    deg = dacc_ref[0] + dacc_ref[1]
    mean = (acc_ref[0, :, :d] + acc_ref[1, :, :d]) / jnp.maximum(deg, 1.0)
    z = mean + r_ref[...]
    m = jnp.max(z, axis=1, keepdims=True)
    ez = jnp.exp(z - m)
    lse = jnp.log(jnp.sum(ez, axis=1, keepdims=True)) + m
    o_ref[...] = z - lse


def _final(acc, dacc, r):
    d = r.shape[1]
    return pl.pallas_call(
        _final_body,
        grid=(GRID,),
        in_specs=[
            pl.BlockSpec((NCORES, BLK, acc.shape[2]), lambda i: (0, i, 0)),
            pl.BlockSpec((NCORES, BLK, 1), lambda i: (0, i, 0)),
            pl.BlockSpec((BLK, d), lambda i: (i, 0)),
        ],
        out_specs=pl.BlockSpec((BLK, d), lambda i: (i, 0)),
        out_shape=jax.ShapeDtypeStruct((N_PAD, d), jnp.float32),
    )(acc, dacc, r)


@jax.jit
def kernel(x, edge_index, Wl1, bl1, Wr1, Wl2, bl2, Wr2, Wl3, bl3, Wr3):
    x_pad = jnp.zeros((N_PAD, D_IN), jnp.float32).at[:N].set(x)
    pad_e = E_PAD + 2 * CH - E
    pad_dst = N + jnp.arange(pad_e, dtype=jnp.int32) % (N_PAD - N)
    src = jnp.concatenate([edge_index[0], jnp.zeros((pad_e,), jnp.int32)])
    dst = jnp.concatenate([edge_index[1], pad_dst])

    dacc = _deg_k(src, dst).reshape(NCORES, N_PAD, D_DEG)[:, :, :1]

    p1, r1 = _pre(x_pad, Wl1.T, Wr1.T, bl1[None])
    acc1 = _agg_h(p1, src, dst).reshape(NCORES, N_PAD, D_H)
    p2, r2 = _mid(acc1, dacc, r1, Wl2.T, Wr2.T, bl2[None])
    acc2 = _agg_h(p2, src, dst).reshape(NCORES, N_PAD, D_H)
    wl3T_pad = jnp.pad(Wl3.T, ((0, 0), (0, D_H - D_OUT)))
    p3, r3 = _mid(acc2, dacc, r2, wl3T_pad, Wr3.T, bl3[None])
    acc3 = _agg_h(p3, src, dst).reshape(NCORES, N_PAD, D_H)
    out = _final(acc3, dacc, r3)
    return out[:N]
